# Initial kernel scaffold; baseline (speedup 1.0000x reference)
import jax
import jax.numpy as jnp
from jax.experimental import pallas as pl


def kernel(inputs):
    raise NotImplementedError("write your pallas kernel here")


# speedup vs baseline: 3.9036x; 3.9036x over previous
import jax
import jax.numpy as jnp
from jax import lax
from jax.experimental import pallas as pl


_N = 7056


def _body(x_ref, o_ref):
    v = x_ref[...]
    m = jnp.max(v)
    ridx = (lax.broadcasted_iota(jnp.int32, (84, 84), 0) * 84
            + lax.broadcasted_iota(jnp.int32, (84, 84), 1))
    cand = jnp.where(v == m, ridx, jnp.int32(_N))
    amin = jnp.min(cand)
    oidx = lax.broadcasted_iota(jnp.int32, (1, _N), 1)
    o_ref[...] = (oidx == amin).astype(jnp.float32)


def kernel(inputs):
    return pl.pallas_call(
        _body,
        out_shape=jax.ShapeDtypeStruct((1, _N), jnp.float32),
    )(inputs)
